# baseline (speedup 1.0000x reference)
import functools

import jax
import jax.numpy as jnp
from jax import lax
from jax.experimental import pallas as pl
from jax.experimental.pallas import tpu as pltpu
from jax.experimental.pallas import tpu_sc as plsc

B, L = 64, 25
V, H, D = 30522, 768, 2048
MM, R = 700, 10


def _sc_gather(table, idx, n_pad, b_per_w, nc):
    mesh = plsc.VectorSubcoreMesh(core_axis_name="c", subcore_axis_name="s")

    @functools.partial(
        pl.kernel, mesh=mesh,
        out_type=jax.ShapeDtypeStruct((n_pad, H), jnp.float32),
        scratch_types=[
            pltpu.VMEM((b_per_w,), jnp.int32),
            pltpu.VMEM((b_per_w, H), jnp.float32),
            pltpu.SemaphoreType.DMA,
        ],
    )
    def k(table_hbm, idx_hbm, out_hbm, idx_v, rows_v, sem):
        wid = lax.axis_index("s") * nc + lax.axis_index("c")
        base = wid * b_per_w
        pltpu.sync_copy(idx_hbm.at[pl.ds(base, b_per_w)], idx_v)
        pltpu.async_copy(table_hbm.at[idx_v], rows_v, sem).wait()
        pltpu.sync_copy(rows_v, out_hbm.at[pl.ds(base, b_per_w)])

    return k(table, idx)


def _x1_body(ve_ref, wv_ref, bv_ref, w1_ref, b1_ref, out_ref):
    l = pl.program_id(0)
    vproj = lax.dot_general(ve_ref[...], wv_ref[...],
                            (((1,), (1,)), ((), ())),
                            preferred_element_type=jnp.float32) + bv_ref[...]
    contrib = lax.dot_general(vproj, w1_ref[...],
                              (((1,), (1,)), ((), ())),
                              preferred_element_type=jnp.float32)

    @pl.when(l == 0)
    def _():
        out_ref[...] = contrib + b1_ref[...]

    @pl.when(l > 0)
    def _():
        out_ref[...] += contrib


def _x1_call(ve, Wv, bv2, W1r, b12):
    return pl.pallas_call(
        _x1_body,
        grid=(L,),
        in_specs=[
            pl.BlockSpec((B, D), lambda l: (0, l)),
            pl.BlockSpec((H, D), lambda l: (0, 0)),
            pl.BlockSpec((1, H), lambda l: (0, 0)),
            pl.BlockSpec((MM, H), lambda l: (0, l)),
            pl.BlockSpec((1, MM), lambda l: (0, 0)),
        ],
        out_specs=pl.BlockSpec((B, MM), lambda l: (0, 0)),
        out_shape=jax.ShapeDtypeStruct((B, MM), jnp.float32),
        compiler_params=pltpu.CompilerParams(
            dimension_semantics=("arbitrary",)),
    )(ve, Wv, bv2, W1r, b12)


def _x0_body(emb_ref, w0_ref, b0_ref, out_ref):
    l = pl.program_id(0)
    contrib = lax.dot_general(emb_ref[...], w0_ref[...],
                              (((1,), (1,)), ((), ())),
                              preferred_element_type=jnp.float32)

    @pl.when(l == 0)
    def _():
        out_ref[...] = contrib + b0_ref[...]

    @pl.when(l > 0)
    def _():
        out_ref[...] += contrib


def _x0_call(emb, W0r, b02):
    return pl.pallas_call(
        _x0_body,
        grid=(L,),
        in_specs=[
            pl.BlockSpec((B, H), lambda l: (0, l)),
            pl.BlockSpec((MM, H), lambda l: (0, l)),
            pl.BlockSpec((1, MM), lambda l: (0, 0)),
        ],
        out_specs=pl.BlockSpec((B, MM), lambda l: (0, 0)),
        out_shape=jax.ShapeDtypeStruct((B, MM), jnp.float32),
        compiler_params=pltpu.CompilerParams(
            dimension_semantics=("arbitrary",)),
    )(emb, W0r, b02)


def _tail_body(x0_ref, x1_ref, wm0_ref, bm0_ref, wm1_ref, bm1_ref,
               wout_ref, bout_ref, lng_ref, lnb_ref, out_ref, acc_ref):
    r = pl.program_id(0)
    m0 = lax.dot_general(x0_ref[...], wm0_ref[0],
                         (((1,), (1,)), ((), ())),
                         preferred_element_type=jnp.float32) + bm0_ref[0]
    m1 = lax.dot_general(x1_ref[...], wm1_ref[0],
                         (((1,), (1,)), ((), ())),
                         preferred_element_type=jnp.float32) + bm1_ref[0]
    prod = m0 * m1

    @pl.when(r == 0)
    def _():
        acc_ref[...] = prod

    @pl.when(r > 0)
    def _():
        acc_ref[...] += prod

    @pl.when(r == R - 1)
    def _():
        z = acc_ref[...]
        y = lax.dot_general(z, wout_ref[...],
                            (((1,), (1,)), ((), ())),
                            preferred_element_type=jnp.float32) + bout_ref[...]
        mu = jnp.mean(y, axis=-1, keepdims=True)
        var = jnp.mean((y - mu) ** 2, axis=-1, keepdims=True)
        out_ref[...] = (y - mu) * lax.rsqrt(var + 1e-12) * lng_ref[...] \
            + lnb_ref[...]


def _tail_call(x0, x1, Wm0r, bm0r, Wm1r, bm1r, Wout, bout2, lng2, lnb2):
    return pl.pallas_call(
        _tail_body,
        grid=(R,),
        in_specs=[
            pl.BlockSpec((B, MM), lambda r: (0, 0)),
            pl.BlockSpec((B, MM), lambda r: (0, 0)),
            pl.BlockSpec((1, MM, MM), lambda r: (r, 0, 0)),
            pl.BlockSpec((1, 1, MM), lambda r: (r, 0, 0)),
            pl.BlockSpec((1, MM, MM), lambda r: (r, 0, 0)),
            pl.BlockSpec((1, 1, MM), lambda r: (r, 0, 0)),
            pl.BlockSpec((H, MM), lambda r: (0, 0)),
            pl.BlockSpec((1, H), lambda r: (0, 0)),
            pl.BlockSpec((1, H), lambda r: (0, 0)),
            pl.BlockSpec((1, H), lambda r: (0, 0)),
        ],
        out_specs=pl.BlockSpec((B, H), lambda r: (0, 0)),
        out_shape=jax.ShapeDtypeStruct((B, H), jnp.float32),
        scratch_shapes=[pltpu.VMEM((B, MM), jnp.float32)],
        compiler_params=pltpu.CompilerParams(
            dimension_semantics=("arbitrary",)),
    )(x0, x1, Wm0r, bm0r, Wm1r, bm1r, Wout, bout2, lng2, lnb2)


def kernel(input_ids, token_type_ids, visual_embeds, visual_token_type_ids,
           word_emb, pos_emb, tt_emb, vtt_emb, vpos_emb, Wv, bv,
           W0, b0, W1, b1, Wm0, bm0, Wm1, bm1, Wout, bout, ln_g, ln_b):
    info = plsc.get_sparse_core_info()
    nc, ns = info.num_cores, info.num_subcores
    nw = nc * ns
    n_tok = B * L
    align = 8 * nw
    n_pad = ((n_tok + align - 1) // align) * align
    b_per_w = n_pad // nw

    ids = input_ids.reshape(-1).astype(jnp.int32)
    ids = jnp.concatenate(
        [ids, jnp.zeros((n_pad - n_tok,), dtype=jnp.int32)])

    rows = _sc_gather(word_emb, ids, n_pad, b_per_w, nc)
    emb = rows[:n_tok].reshape(B, L * H)

    x1 = _x1_call(visual_embeds.reshape(B, L * D), Wv, bv.reshape(1, H),
                  W1, b1.reshape(1, MM))
    x0 = _x0_call(emb, W0, b0.reshape(1, MM))

    return _tail_call(x0, x1,
                      Wm0.reshape(R, MM, MM), bm0.reshape(R, 1, MM),
                      Wm1.reshape(R, MM, MM), bm1.reshape(R, 1, MM),
                      Wout, bout.reshape(1, H),
                      ln_g.reshape(1, H), ln_b.reshape(1, H))


# speedup vs baseline: 1.0166x; 1.0166x over previous
import functools

import jax
import jax.numpy as jnp
from jax import lax
from jax.experimental import pallas as pl
from jax.experimental.pallas import tpu as pltpu
from jax.experimental.pallas import tpu_sc as plsc

B, L = 64, 25
V, H, D = 30522, 768, 2048
MM, R = 700, 10


def _sc_gather(table, idx, n_pad, b_per_w, nc):
    mesh = plsc.VectorSubcoreMesh(core_axis_name="c", subcore_axis_name="s")

    @functools.partial(
        pl.kernel, mesh=mesh,
        out_type=jax.ShapeDtypeStruct((n_pad, H), jnp.float32),
        scratch_types=[
            pltpu.VMEM((b_per_w,), jnp.int32),
            pltpu.VMEM((b_per_w, H), jnp.float32),
            pltpu.SemaphoreType.DMA,
        ],
        compiler_params=pltpu.CompilerParams(use_tc_tiling_on_sc=True),
    )
    def k(table_hbm, idx_hbm, out_hbm, idx_v, rows_v, sem):
        wid = lax.axis_index("s") * nc + lax.axis_index("c")
        base = wid * b_per_w
        pltpu.sync_copy(idx_hbm.at[pl.ds(base, b_per_w)], idx_v)
        pltpu.async_copy(table_hbm.at[idx_v], rows_v, sem).wait()
        pltpu.sync_copy(rows_v, out_hbm.at[pl.ds(base, b_per_w)])

    return k(table, idx)


def _x1_body(ve_ref, wv_ref, bv_ref, w1_ref, b1_ref, out_ref):
    l = pl.program_id(0)
    vproj = lax.dot_general(ve_ref[...], wv_ref[...],
                            (((1,), (1,)), ((), ())),
                            preferred_element_type=jnp.float32) + bv_ref[...]
    contrib = lax.dot_general(vproj, w1_ref[...],
                              (((1,), (1,)), ((), ())),
                              preferred_element_type=jnp.float32)

    @pl.when(l == 0)
    def _():
        out_ref[...] = contrib + b1_ref[...]

    @pl.when(l > 0)
    def _():
        out_ref[...] += contrib


def _x1_call(ve, Wv, bv2, W1r, b12):
    return pl.pallas_call(
        _x1_body,
        grid=(L,),
        in_specs=[
            pl.BlockSpec((B, D), lambda l: (0, l)),
            pl.BlockSpec((H, D), lambda l: (0, 0)),
            pl.BlockSpec((1, H), lambda l: (0, 0)),
            pl.BlockSpec((MM, H), lambda l: (0, l)),
            pl.BlockSpec((1, MM), lambda l: (0, 0)),
        ],
        out_specs=pl.BlockSpec((B, MM), lambda l: (0, 0)),
        out_shape=jax.ShapeDtypeStruct((B, MM), jnp.float32),
        compiler_params=pltpu.CompilerParams(
            dimension_semantics=("arbitrary",)),
    )(ve, Wv, bv2, W1r, b12)


def _x0_body(emb_ref, w0_ref, b0_ref, out_ref):
    l = pl.program_id(0)
    contrib = lax.dot_general(emb_ref[...], w0_ref[...],
                              (((1,), (1,)), ((), ())),
                              preferred_element_type=jnp.float32)

    @pl.when(l == 0)
    def _():
        out_ref[...] = contrib + b0_ref[...]

    @pl.when(l > 0)
    def _():
        out_ref[...] += contrib


def _x0_call(emb, W0r, b02):
    return pl.pallas_call(
        _x0_body,
        grid=(L,),
        in_specs=[
            pl.BlockSpec((B, H), lambda l: (0, l)),
            pl.BlockSpec((MM, H), lambda l: (0, l)),
            pl.BlockSpec((1, MM), lambda l: (0, 0)),
        ],
        out_specs=pl.BlockSpec((B, MM), lambda l: (0, 0)),
        out_shape=jax.ShapeDtypeStruct((B, MM), jnp.float32),
        compiler_params=pltpu.CompilerParams(
            dimension_semantics=("arbitrary",)),
    )(emb, W0r, b02)


def _tail_body(x0_ref, x1_ref, wm0_ref, bm0_ref, wm1_ref, bm1_ref,
               wout_ref, bout_ref, lng_ref, lnb_ref, out_ref, acc_ref):
    r = pl.program_id(0)
    m0 = lax.dot_general(x0_ref[...], wm0_ref[0],
                         (((1,), (1,)), ((), ())),
                         preferred_element_type=jnp.float32) + bm0_ref[0]
    m1 = lax.dot_general(x1_ref[...], wm1_ref[0],
                         (((1,), (1,)), ((), ())),
                         preferred_element_type=jnp.float32) + bm1_ref[0]
    prod = m0 * m1

    @pl.when(r == 0)
    def _():
        acc_ref[...] = prod

    @pl.when(r > 0)
    def _():
        acc_ref[...] += prod

    @pl.when(r == R - 1)
    def _():
        z = acc_ref[...]
        y = lax.dot_general(z, wout_ref[...],
                            (((1,), (1,)), ((), ())),
                            preferred_element_type=jnp.float32) + bout_ref[...]
        mu = jnp.mean(y, axis=-1, keepdims=True)
        var = jnp.mean((y - mu) ** 2, axis=-1, keepdims=True)
        out_ref[...] = (y - mu) * lax.rsqrt(var + 1e-12) * lng_ref[...] \
            + lnb_ref[...]


def _tail_call(x0, x1, Wm0r, bm0r, Wm1r, bm1r, Wout, bout2, lng2, lnb2):
    return pl.pallas_call(
        _tail_body,
        grid=(R,),
        in_specs=[
            pl.BlockSpec((B, MM), lambda r: (0, 0)),
            pl.BlockSpec((B, MM), lambda r: (0, 0)),
            pl.BlockSpec((1, MM, MM), lambda r: (r, 0, 0)),
            pl.BlockSpec((1, 1, MM), lambda r: (r, 0, 0)),
            pl.BlockSpec((1, MM, MM), lambda r: (r, 0, 0)),
            pl.BlockSpec((1, 1, MM), lambda r: (r, 0, 0)),
            pl.BlockSpec((H, MM), lambda r: (0, 0)),
            pl.BlockSpec((1, H), lambda r: (0, 0)),
            pl.BlockSpec((1, H), lambda r: (0, 0)),
            pl.BlockSpec((1, H), lambda r: (0, 0)),
        ],
        out_specs=pl.BlockSpec((B, H), lambda r: (0, 0)),
        out_shape=jax.ShapeDtypeStruct((B, H), jnp.float32),
        scratch_shapes=[pltpu.VMEM((B, MM), jnp.float32)],
        compiler_params=pltpu.CompilerParams(
            dimension_semantics=("arbitrary",)),
    )(x0, x1, Wm0r, bm0r, Wm1r, bm1r, Wout, bout2, lng2, lnb2)


def kernel(input_ids, token_type_ids, visual_embeds, visual_token_type_ids,
           word_emb, pos_emb, tt_emb, vtt_emb, vpos_emb, Wv, bv,
           W0, b0, W1, b1, Wm0, bm0, Wm1, bm1, Wout, bout, ln_g, ln_b):
    info = plsc.get_sparse_core_info()
    nc, ns = info.num_cores, info.num_subcores
    nw = nc * ns
    n_tok = B * L
    align = 8 * nw
    n_pad = ((n_tok + align - 1) // align) * align
    b_per_w = n_pad // nw

    ids = input_ids.reshape(-1).astype(jnp.int32)
    ids = jnp.concatenate(
        [ids, jnp.zeros((n_pad - n_tok,), dtype=jnp.int32)])

    rows = _sc_gather(word_emb, ids, n_pad, b_per_w, nc)
    emb = rows[:n_tok].reshape(B, L * H)

    x1 = _x1_call(visual_embeds.reshape(B, L * D), Wv, bv.reshape(1, H),
                  W1, b1.reshape(1, MM))
    x0 = _x0_call(emb, W0, b0.reshape(1, MM))

    return _tail_call(x0, x1,
                      Wm0.reshape(R, MM, MM), bm0.reshape(R, 1, MM),
                      Wm1.reshape(R, MM, MM), bm1.reshape(R, 1, MM),
                      Wout, bout.reshape(1, H),
                      ln_g.reshape(1, H), ln_b.reshape(1, H))
